# Initial kernel scaffold; baseline (speedup 1.0000x reference)
import jax
import jax.numpy as jnp
from jax.experimental import pallas as pl


def kernel(features, position_logits):
    raise NotImplementedError("write your pallas kernel here")


# speedup vs baseline: 3.5439x; 3.5439x over previous
import functools

import jax
import jax.numpy as jnp
from jax.experimental import pallas as pl


def _body(feat_ref, logits_ref, img_ref, conf_ref):
    N = logits_ref.shape[1]
    L = logits_ref[0]
    F = feat_ref[0]

    m = jnp.max(L, axis=0)
    s = jnp.sum(jnp.exp(L - m[None, :]), axis=0)
    conf_ref[0, 0, :] = 1.0 / s

    ii = jax.lax.broadcasted_iota(jnp.int32, (N, N), 0)
    pp = jax.lax.broadcasted_iota(jnp.int32, (N, N), 1)
    preds = jnp.min(jnp.where(L == m[None, :], ii, N), axis=0)

    hit = preds[:, None] == pp
    lastn = jnp.max(jnp.where(hit, ii, -1), axis=0)

    M = (lastn[:, None] == pp).astype(jnp.float32)

    interior = (M[:-2] + M[1:-1] + M[2:]) * (1.0 / 3.0)
    M2 = jnp.concatenate([M[0:1], interior, M[N - 1:N]], axis=0)

    img_ref[0] = jax.lax.dot_general(
        F, M2,
        dimension_numbers=(((0,), (1,)), ((), ())),
        preferred_element_type=jnp.float32,
        precision=jax.lax.Precision.HIGHEST,
    )


@jax.jit
def kernel(features, position_logits):
    B, N, D = features.shape
    img, conf = pl.pallas_call(
        _body,
        grid=(B,),
        in_specs=[
            pl.BlockSpec((1, N, D), lambda b: (b, 0, 0)),
            pl.BlockSpec((1, N, N), lambda b: (b, 0, 0)),
        ],
        out_specs=[
            pl.BlockSpec((1, D, N), lambda b: (b, 0, 0)),
            pl.BlockSpec((1, 1, N), lambda b: (b, 0, 0)),
        ],
        out_shape=[
            jax.ShapeDtypeStruct((B, D, N), jnp.float32),
            jax.ShapeDtypeStruct((B, 1, N), jnp.float32),
        ],
    )(features, position_logits)
    g = int(round(N ** 0.5))
    return img.reshape(B, D, g, g), conf.reshape(B, N)


# speedup vs baseline: 4.0737x; 1.1495x over previous
import functools

import jax
import jax.numpy as jnp
from jax.experimental import pallas as pl


def _body(feat_ref, logits_ref, img_ref, conf_ref):
    N = logits_ref.shape[1]
    L = logits_ref[0]
    F = feat_ref[0]

    m = jnp.max(L, axis=0)
    s = jnp.sum(jnp.exp(L - m[None, :]), axis=0)
    conf_ref[0, 0, :] = 1.0 / s

    ii = jax.lax.broadcasted_iota(jnp.int32, (N, N), 0)
    pp = jax.lax.broadcasted_iota(jnp.int32, (N, N), 1)
    preds = jnp.min(jnp.where(L == m[None, :], ii, N), axis=0)

    hit = preds[:, None] == pp
    lastn = jnp.max(jnp.where(hit, ii, -1), axis=0)

    M = (lastn[:, None] == pp).astype(jnp.float32)

    interior = (M[:-2] + M[1:-1] + M[2:]) * (1.0 / 3.0)
    M2 = jnp.concatenate([M[0:1], interior, M[N - 1:N]], axis=0)

    img_ref[0] = jax.lax.dot_general(
        F.astype(jnp.bfloat16), M2.astype(jnp.bfloat16),
        dimension_numbers=(((0,), (1,)), ((), ())),
        preferred_element_type=jnp.float32,
    )


@jax.jit
def kernel(features, position_logits):
    B, N, D = features.shape
    img, conf = pl.pallas_call(
        _body,
        grid=(B,),
        in_specs=[
            pl.BlockSpec((1, N, D), lambda b: (b, 0, 0)),
            pl.BlockSpec((1, N, N), lambda b: (b, 0, 0)),
        ],
        out_specs=[
            pl.BlockSpec((1, D, N), lambda b: (b, 0, 0)),
            pl.BlockSpec((1, 1, N), lambda b: (b, 0, 0)),
        ],
        out_shape=[
            jax.ShapeDtypeStruct((B, D, N), jnp.float32),
            jax.ShapeDtypeStruct((B, 1, N), jnp.float32),
        ],
    )(features, position_logits)
    g = int(round(N ** 0.5))
    return img.reshape(B, D, g, g), conf.reshape(B, N)


# speedup vs baseline: 5.5202x; 1.3551x over previous
import functools

import jax
import jax.numpy as jnp
from jax.experimental import pallas as pl

_BB = 4


def _body(feat_ref, logits_ref, img_ref, conf_ref):
    N = logits_ref.shape[1]
    ii = jax.lax.broadcasted_iota(jnp.int32, (N, N), 0)
    pp = jax.lax.broadcasted_iota(jnp.int32, (N, N), 1)

    for b in range(_BB):
        L = logits_ref[b]
        F = feat_ref[b]

        m = jnp.max(L, axis=0)
        s = jnp.sum(jnp.exp(L - m[None, :]), axis=0)
        conf_ref[b, 0, :] = 1.0 / s

        preds = jnp.min(jnp.where(L == m[None, :], ii, N), axis=0)

        hit = preds[:, None] == pp
        lastn = jnp.max(jnp.where(hit, ii, -1), axis=0)

        M = (lastn[:, None] == pp).astype(jnp.float32)

        interior = (M[:-2] + M[1:-1] + M[2:]) * (1.0 / 3.0)
        M2 = jnp.concatenate([M[0:1], interior, M[N - 1:N]], axis=0)

        img_ref[b] = jax.lax.dot_general(
            F.astype(jnp.bfloat16), M2.astype(jnp.bfloat16),
            dimension_numbers=(((0,), (1,)), ((), ())),
            preferred_element_type=jnp.float32,
        )


@jax.jit
def kernel(features, position_logits):
    B, N, D = features.shape
    img, conf = pl.pallas_call(
        _body,
        grid=(B // _BB,),
        in_specs=[
            pl.BlockSpec((_BB, N, D), lambda b: (b, 0, 0)),
            pl.BlockSpec((_BB, N, N), lambda b: (b, 0, 0)),
        ],
        out_specs=[
            pl.BlockSpec((_BB, D, N), lambda b: (b, 0, 0)),
            pl.BlockSpec((_BB, 1, N), lambda b: (b, 0, 0)),
        ],
        out_shape=[
            jax.ShapeDtypeStruct((B, D, N), jnp.float32),
            jax.ShapeDtypeStruct((B, 1, N), jnp.float32),
        ],
    )(features, position_logits)
    g = int(round(N ** 0.5))
    return img.reshape(B, D, g, g), conf.reshape(B, N)


# speedup vs baseline: 5.6959x; 1.0318x over previous
import functools

import jax
import jax.numpy as jnp
from jax.experimental import pallas as pl

_BB = 8


def _body(feat_ref, logits_ref, img_ref, conf_ref):
    N = logits_ref.shape[1]
    ii = jax.lax.broadcasted_iota(jnp.int32, (N, N), 0)
    pp = jax.lax.broadcasted_iota(jnp.int32, (N, N), 1)

    for b in range(_BB):
        L = logits_ref[b]
        F = feat_ref[b]

        m = jnp.max(L, axis=0)
        s = jnp.sum(jnp.exp(L - m[None, :]), axis=0)
        conf_ref[b, 0, :] = 1.0 / s

        preds = jnp.min(jnp.where(L == m[None, :], ii, N), axis=0)

        hit = preds[:, None] == pp
        lastn = jnp.max(jnp.where(hit, ii, -1), axis=0)

        M = (lastn[:, None] == pp).astype(jnp.float32)

        interior = (M[:-2] + M[1:-1] + M[2:]) * (1.0 / 3.0)
        M2 = jnp.concatenate([M[0:1], interior, M[N - 1:N]], axis=0)

        img_ref[b] = jax.lax.dot_general(
            F.astype(jnp.bfloat16), M2.astype(jnp.bfloat16),
            dimension_numbers=(((0,), (1,)), ((), ())),
            preferred_element_type=jnp.float32,
        )


@jax.jit
def kernel(features, position_logits):
    B, N, D = features.shape
    img, conf = pl.pallas_call(
        _body,
        grid=(B // _BB,),
        in_specs=[
            pl.BlockSpec((_BB, N, D), lambda b: (b, 0, 0)),
            pl.BlockSpec((_BB, N, N), lambda b: (b, 0, 0)),
        ],
        out_specs=[
            pl.BlockSpec((_BB, D, N), lambda b: (b, 0, 0)),
            pl.BlockSpec((_BB, 1, N), lambda b: (b, 0, 0)),
        ],
        out_shape=[
            jax.ShapeDtypeStruct((B, D, N), jnp.float32),
            jax.ShapeDtypeStruct((B, 1, N), jnp.float32),
        ],
    )(features, position_logits)
    g = int(round(N ** 0.5))
    return img.reshape(B, D, g, g), conf.reshape(B, N)
